# baseline (speedup 1.0000x reference)
import functools

import jax
import jax.numpy as jnp
from jax import lax
from jax.experimental import pallas as pl
from jax.experimental.pallas import tpu as pltpu
from jax.experimental.pallas import tpu_sc as plsc

_BATCH = 16384
_DIM = 64
_NC = 2
_NS = 16
_NW = _NC * _NS
_BPW = _BATCH // _NW
_CHUNK = 256
_NCHUNK = _BPW // _CHUNK
_CGRP = _CHUNK // 16


def _rsqrt(x):
    i = plsc.bitcast(x, jnp.int32)
    i = jnp.int32(0x5F3759DF) - (i >> 1)
    y = plsc.bitcast(i, jnp.float32)
    xh = 0.5 * x
    for _ in range(3):
        y = y * (1.5 - xh * y * y)
    return y


_mesh = plsc.VectorSubcoreMesh(core_axis_name="c", subcore_axis_name="s")


@functools.partial(
    pl.kernel,
    mesh=_mesh,
    out_type=jax.ShapeDtypeStruct((_BATCH,), jnp.float32),
    compiler_params=pltpu.CompilerParams(
        needs_layout_passes=False, use_tc_tiling_on_sc=True),
    scratch_types=[
        pltpu.VMEM((_BPW * 3,), jnp.int32),
        pltpu.VMEM((_CHUNK, _DIM), jnp.float32),
        pltpu.VMEM((_CHUNK, _DIM), jnp.float32),
        pltpu.VMEM((_CHUNK, _DIM), jnp.float32),
        pltpu.VMEM((_BPW,), jnp.float32),
        pltpu.SemaphoreType.DMA,
    ],
)
def _transe_sc(paths_hbm, node_hbm, rel_hbm, out_hbm,
               paths_v, hbuf, rbuf, tbuf, out_v, sem):
    wid = lax.axis_index("s") * _NC + lax.axis_index("c")
    base = wid * _BPW

    pltpu.sync_copy(paths_hbm.at[pl.ds(base * 3, _BPW * 3)], paths_v)

    iota16 = lax.iota(jnp.int32, 16)
    iota48 = iota16 * 3

    def _drain_16(i):
        for j in range(16):
            k = i * 16 + j
            pltpu.make_async_copy(node_hbm.at[0], hbuf.at[k], sem).wait()
            pltpu.make_async_copy(rel_hbm.at[0], rbuf.at[k], sem).wait()
            pltpu.make_async_copy(node_hbm.at[0], tbuf.at[k], sem).wait()

    for c in range(_NCHUNK):
        def fire_body(i, carry, _c=c):
            p = iota48 + (_c * _CHUNK * 3 + i * 48)
            t16 = plsc.load_gather(paths_v, [p])
            r16 = plsc.load_gather(paths_v, [p + 1])
            h16 = plsc.load_gather(paths_v, [p + 2])
            for j in range(16):
                k = i * 16 + j
                pltpu.async_copy(node_hbm.at[h16[j]], hbuf.at[k], sem)
                pltpu.async_copy(rel_hbm.at[r16[j]], rbuf.at[k], sem)
                pltpu.async_copy(node_hbm.at[t16[j]], tbuf.at[k], sem)

            @pl.when(i > 0)
            def _():
                _drain_16(i - 1)

            return carry

        lax.fori_loop(0, _CGRP, fire_body, 0)
        _drain_16(_CGRP - 1)

        def group_body(i, carry, _c=c):
            lrows = iota16 + i * 16

            def col_body(cc, acc):
                hh, tt, rr, hr, ht, rt = acc
                cs = jnp.full((16,), 0, jnp.int32) + cc
                h = plsc.load_gather(hbuf, [lrows, cs])
                r = plsc.load_gather(rbuf, [lrows, cs])
                t = plsc.load_gather(tbuf, [lrows, cs])
                return (hh + h * h, tt + t * t, rr + r * r,
                        hr + h * r, ht + h * t, rt + r * t)

            z = jnp.full((16,), 0.0, jnp.float32)
            hh, tt, rr, hr, ht, rt = lax.fori_loop(
                0, _DIM, col_body, (z, z, z, z, z, z), unroll=8)

            a = _rsqrt(jnp.maximum(hh, 1e-24))
            b = _rsqrt(jnp.maximum(tt, 1e-24))
            dd = (hh * a * a + rr + tt * b * b
                  + 2.0 * (a * hr - (a * b) * ht - b * rt))
            ddc = jnp.maximum(dd, 1e-30)
            out_v[pl.ds(_c * _CHUNK + i * 16, 16)] = -(ddc * _rsqrt(ddc))
            return carry

        lax.fori_loop(0, _CGRP, group_body, 0)

    pltpu.sync_copy(out_v, out_hbm.at[pl.ds(base, _BPW)])


def kernel(batched_paths, node_emb, rel_emb):
    return _transe_sc(batched_paths.reshape(-1), node_emb, rel_emb)
